# Initial kernel scaffold; baseline (speedup 1.0000x reference)
import jax
import jax.numpy as jnp
from jax.experimental import pallas as pl


def kernel(x, token_table, position_table):
    raise NotImplementedError("write your pallas kernel here")


# speedup vs baseline: 2.2886x; 2.2886x over previous
import functools

import jax
import jax.numpy as jnp
from jax import lax
from jax.experimental import pallas as pl
from jax.experimental.pallas import tpu as pltpu
from jax.experimental.pallas import tpu_sc as plsc

_NC = 2
_NS = 16
_NW = _NC * _NS
_CHUNK = 128


def _emb_body(nchunk, seq_len, embed, x_hbm, tok_hbm, pos_hbm, out_hbm,
              idx_v, buf, posb, sem):
    wid = lax.axis_index("s") * _NC + lax.axis_index("c")
    rows_per_w = nchunk * _CHUNK
    base = wid * rows_per_w
    s_base = lax.rem(base, seq_len)
    pltpu.sync_copy(x_hbm.at[wid], idx_v)
    for c in range(nchunk):
        row0 = base + c * _CHUNK
        pltpu.async_copy(tok_hbm.at[idx_v.at[c]], buf, sem).wait()
        pltpu.sync_copy(pos_hbm.at[pl.ds(s_base + c * _CHUNK, _CHUNK)], posb)

        def add_row(i, carry):
            for j in range(embed // 16):
                sl = pl.ds(j * 16, 16)
                plsc.addupdate(buf.at[i, sl], posb[i, sl])
            return carry

        lax.fori_loop(0, _CHUNK, add_row, 0)
        pltpu.sync_copy(buf, out_hbm.at[pl.ds(row0, _CHUNK)])


def kernel(x, token_table, position_table):
    b, s = x.shape
    vocab, embed = token_table.shape
    n = b * s
    assert n % (_NW * _CHUNK) == 0 and embed % 16 == 0
    rows_per_w = n // _NW
    nchunk = rows_per_w // _CHUNK
    assert s % rows_per_w == 0 or rows_per_w % s == 0

    x3 = x.reshape(_NW, nchunk, _CHUNK).astype(jnp.int32)

    mesh = plsc.VectorSubcoreMesh(core_axis_name="c", subcore_axis_name="s")
    body = functools.partial(_emb_body, nchunk, s, embed)
    out = pl.kernel(
        body,
        mesh=mesh,
        out_type=jax.ShapeDtypeStruct((n, embed), jnp.float32),
        scratch_types=[
            pltpu.VMEM((nchunk, _CHUNK), jnp.int32),
            pltpu.VMEM((_CHUNK, embed), jnp.float32),
            pltpu.VMEM((_CHUNK, embed), jnp.float32),
            pltpu.SemaphoreType.DMA,
        ],
    )(x3, token_table, position_table)
    return out.reshape(b, s, embed)


# speedup vs baseline: 2.8677x; 1.2530x over previous
import functools

import jax
import jax.numpy as jnp
from jax import lax
from jax.experimental import pallas as pl
from jax.experimental.pallas import tpu as pltpu
from jax.experimental.pallas import tpu_sc as plsc

_NC = 2
_NS = 16
_NW = _NC * _NS
_CHUNK = 128
_RUNROLL = 4


def _emb_body(nchunk, seq_len, embed, x_hbm, tok_hbm, pos_hbm, out_hbm,
              idx_v, buf0, buf1, posb0, posb1, sem0, sem1):
    bufs = (buf0, buf1)
    posbs = (posb0, posb1)
    sems = (sem0, sem1)
    wid = lax.axis_index("s") * _NC + lax.axis_index("c")
    rows_per_w = nchunk * _CHUNK
    base = wid * rows_per_w
    s_base = lax.rem(base, seq_len)
    pltpu.sync_copy(x_hbm.at[wid], idx_v)

    def start(c):
        slot = c & 1
        g = pltpu.async_copy(tok_hbm.at[idx_v.at[c]], bufs[slot], sems[slot])
        p = pltpu.async_copy(
            pos_hbm.at[pl.ds(s_base + c * _CHUNK, _CHUNK)], posbs[slot],
            sems[slot])
        return g, p

    pending = start(0)
    for c in range(nchunk):
        slot = c & 1
        nxt = pending if c + 1 >= nchunk else start(c + 1)
        g, p = pending
        g.wait()
        p.wait()
        pending = nxt
        buf, posb = bufs[slot], posbs[slot]

        def add_rows(i, carry):
            for r in range(_RUNROLL):
                row = i * _RUNROLL + r
                for j in range(embed // 16):
                    sl = pl.ds(j * 16, 16)
                    plsc.addupdate(buf.at[row, sl], posb[row, sl])
            return carry

        lax.fori_loop(0, _CHUNK // _RUNROLL, add_rows, 0)
        pltpu.sync_copy(buf, out_hbm.at[pl.ds(base + c * _CHUNK, _CHUNK)])


def kernel(x, token_table, position_table):
    b, s = x.shape
    vocab, embed = token_table.shape
    n = b * s
    assert n % (_NW * _CHUNK) == 0 and embed % 16 == 0
    rows_per_w = n // _NW
    nchunk = rows_per_w // _CHUNK
    assert s % rows_per_w == 0 or rows_per_w % s == 0

    x3 = x.reshape(_NW, nchunk, _CHUNK).astype(jnp.int32)

    mesh = plsc.VectorSubcoreMesh(core_axis_name="c", subcore_axis_name="s")
    body = functools.partial(_emb_body, nchunk, s, embed)
    out = pl.kernel(
        body,
        mesh=mesh,
        out_type=jax.ShapeDtypeStruct((n, embed), jnp.float32),
        scratch_types=[
            pltpu.VMEM((nchunk, _CHUNK), jnp.int32),
            pltpu.VMEM((_CHUNK, embed), jnp.float32),
            pltpu.VMEM((_CHUNK, embed), jnp.float32),
            pltpu.VMEM((_CHUNK, embed), jnp.float32),
            pltpu.VMEM((_CHUNK, embed), jnp.float32),
            pltpu.SemaphoreType.DMA,
            pltpu.SemaphoreType.DMA,
        ],
    )(x3, token_table, position_table)
    return out.reshape(b, s, embed)


# speedup vs baseline: 3.0656x; 1.0690x over previous
import functools

import jax
import jax.numpy as jnp
from jax import lax
from jax.experimental import pallas as pl
from jax.experimental.pallas import tpu as pltpu
from jax.experimental.pallas import tpu_sc as plsc

_NC = 2
_NS = 16
_NW = _NC * _NS
_CHUNK = 128


def _emb_body(nchunk, seq_len, embed, x_hbm, tok_hbm, pos_hbm, out_hbm,
              idx_v, buf0, buf1, psem0, psem1, gsem0, gsem1, wsem0, wsem1):
    bufs = (buf0, buf1)
    psems = (psem0, psem1)
    gsems = (gsem0, gsem1)
    wsems = (wsem0, wsem1)
    wid = lax.axis_index("s") * _NC + lax.axis_index("c")
    rows_per_w = nchunk * _CHUNK
    base = wid * rows_per_w
    s_base = lax.rem(base, seq_len)
    pltpu.sync_copy(x_hbm.at[wid], idx_v)

    def posfill(c):
        slot = c & 1
        return pltpu.async_copy(
            pos_hbm.at[pl.ds(s_base + c * _CHUNK, _CHUNK)], bufs[slot],
            psems[slot])

    fill = posfill(0)
    wb = None
    for c in range(nchunk):
        slot = c & 1
        fill.wait()
        gadd = pltpu.async_copy(tok_hbm.at[idx_v.at[c]], bufs[slot],
                                gsems[slot], add=True)
        if c + 1 < nchunk:
            if wb is not None:
                wb.wait()
            fill = posfill(c + 1)
        gadd.wait()
        wb = pltpu.async_copy(bufs[slot], out_hbm.at[pl.ds(base + c * _CHUNK,
                                                           _CHUNK)],
                              wsems[slot])
    wb.wait()


def kernel(x, token_table, position_table):
    b, s = x.shape
    vocab, embed = token_table.shape
    n = b * s
    assert n % (_NW * _CHUNK) == 0 and embed % 16 == 0
    rows_per_w = n // _NW
    nchunk = rows_per_w // _CHUNK
    assert s % rows_per_w == 0 or rows_per_w % s == 0

    x3 = x.reshape(_NW, nchunk, _CHUNK).astype(jnp.int32)

    mesh = plsc.VectorSubcoreMesh(core_axis_name="c", subcore_axis_name="s")
    body = functools.partial(_emb_body, nchunk, s, embed)
    out = pl.kernel(
        body,
        mesh=mesh,
        out_type=jax.ShapeDtypeStruct((n, embed), jnp.float32),
        scratch_types=[
            pltpu.VMEM((nchunk, _CHUNK), jnp.int32),
            pltpu.VMEM((_CHUNK, embed), jnp.float32),
            pltpu.VMEM((_CHUNK, embed), jnp.float32),
            pltpu.SemaphoreType.DMA,
            pltpu.SemaphoreType.DMA,
            pltpu.SemaphoreType.DMA,
            pltpu.SemaphoreType.DMA,
            pltpu.SemaphoreType.DMA,
            pltpu.SemaphoreType.DMA,
        ],
    )(x3, token_table, position_table)
    return out.reshape(b, s, embed)


# speedup vs baseline: 3.1166x; 1.0166x over previous
import functools

import jax
import jax.numpy as jnp
from jax import lax
from jax.experimental import pallas as pl
from jax.experimental.pallas import tpu as pltpu
from jax.experimental.pallas import tpu_sc as plsc

_NC = 2
_NS = 16
_NW = _NC * _NS
_RUNROLL = 4


def _emb_body(nb, seq_len, embed, x_hbm, tok_hbm, pos_hbm, out_hbm,
              idx_v, posb, buf0, buf1, gsem0, gsem1, wsem0, wsem1):
    chunk = seq_len // _NW
    bufs = (buf0, buf1)
    gsems = (gsem0, gsem1)
    wsems = (wsem0, wsem1)
    wid = lax.axis_index("s") * _NC + lax.axis_index("c")
    s_base = wid * chunk
    pltpu.sync_copy(pos_hbm.at[pl.ds(s_base, chunk)], posb)
    pltpu.sync_copy(x_hbm.at[wid], idx_v)

    def gather(b):
        slot = b & 1
        return pltpu.async_copy(tok_hbm.at[idx_v.at[b]], bufs[slot],
                                gsems[slot])

    g = gather(0)
    wb = None
    for b in range(nb):
        slot = b & 1
        buf = bufs[slot]
        g.wait()
        if b + 1 < nb:
            if wb is not None:
                wb.wait()
            g = gather(b + 1)

        def add_rows(i, carry):
            for r in range(_RUNROLL):
                row = i * _RUNROLL + r
                for j in range(embed // 16):
                    sl = pl.ds(j * 16, 16)
                    plsc.addupdate(buf.at[row, sl], posb[row, sl])
            return carry

        lax.fori_loop(0, chunk // _RUNROLL, add_rows, 0)
        wb = pltpu.async_copy(
            buf, out_hbm.at[pl.ds(b * seq_len + s_base, chunk)],
            wsems[slot])
    wb.wait()


def kernel(x, token_table, position_table):
    b, s = x.shape
    vocab, embed = token_table.shape
    n = b * s
    chunk = s // _NW
    assert s % _NW == 0 and embed % 16 == 0 and chunk <= 128
    assert chunk % _RUNROLL == 0

    xw = jnp.transpose(x.astype(jnp.int32)).reshape(_NW, chunk, b)
    xw = jnp.transpose(xw, (0, 2, 1))

    mesh = plsc.VectorSubcoreMesh(core_axis_name="c", subcore_axis_name="s")
    body = functools.partial(_emb_body, b, s, embed)
    out = pl.kernel(
        body,
        mesh=mesh,
        out_type=jax.ShapeDtypeStruct((n, embed), jnp.float32),
        scratch_types=[
            pltpu.VMEM((b, chunk), jnp.int32),
            pltpu.VMEM((chunk, embed), jnp.float32),
            pltpu.VMEM((chunk, embed), jnp.float32),
            pltpu.VMEM((chunk, embed), jnp.float32),
            pltpu.SemaphoreType.DMA,
            pltpu.SemaphoreType.DMA,
            pltpu.SemaphoreType.DMA,
            pltpu.SemaphoreType.DMA,
        ],
    )(xw, token_table, position_table)
    return out.reshape(b, s, embed)


# speedup vs baseline: 3.4682x; 1.1128x over previous
import functools

import jax
import jax.numpy as jnp
from jax import lax
from jax.experimental import pallas as pl
from jax.experimental.pallas import tpu as pltpu
from jax.experimental.pallas import tpu_sc as plsc

_NC = 2
_NS = 16
_NW = _NC * _NS
_CHUNK = 128
_RUNROLL = 4
_NBUF = 3


def _emb_body(nb, seq_len, embed, x_hbm, tok_hbm, pos_hbm, out_hbm,
              idx_v, posb, buf0, buf1, buf2,
              gsem0, gsem1, gsem2, wsem0, wsem1, wsem2):
    bufs = (buf0, buf1, buf2)
    gsems = (gsem0, gsem1, gsem2)
    wsems = (wsem0, wsem1, wsem2)
    wid = lax.axis_index("s") * _NC + lax.axis_index("c")
    gb = wid // _NS
    gs = lax.rem(wid, _NS)
    s_base = gs * _CHUNK
    pltpu.sync_copy(pos_hbm.at[pl.ds(s_base, _CHUNK)], posb)
    pltpu.sync_copy(x_hbm.at[wid], idx_v)

    def gather(k):
        return pltpu.async_copy(tok_hbm.at[idx_v.at[k]], bufs[k % _NBUF],
                                gsems[k % _NBUF])

    gs_pend = [gather(0), gather(1)]
    wbs = [None] * nb
    for k in range(nb):
        buf = bufs[k % _NBUF]
        gs_pend[k].wait()
        if k + 2 < nb:
            if k >= 1:
                wbs[k - 1].wait()
            gs_pend.append(gather(k + 2))

        @plsc.parallel_loop(0, _CHUNK, step=_RUNROLL, unroll=2)
        def add_rows(i):
            for r in range(_RUNROLL):
                for j in range(embed // 16):
                    sl = pl.ds(j * 16, 16)
                    plsc.addupdate(buf.at[i + r, sl], posb[i + r, sl])

        row0 = (gb * nb + k) * seq_len + s_base
        wbs[k] = pltpu.async_copy(buf, out_hbm.at[pl.ds(row0, _CHUNK)],
                                  wsems[k % _NBUF])
    for k in range(max(0, nb - _NBUF), nb):
        wbs[k].wait()


def kernel(x, token_table, position_table):
    b, s = x.shape
    vocab, embed = token_table.shape
    n = b * s
    ns_groups = s // _CHUNK
    nb_groups = _NW // ns_groups
    nb = b // nb_groups
    assert s % _CHUNK == 0 and _NW % ns_groups == 0 and b % nb_groups == 0
    assert embed % 16 == 0 and _CHUNK % _RUNROLL == 0

    xw = (x.astype(jnp.int32)
          .reshape(nb_groups, nb, ns_groups, _CHUNK)
          .transpose(0, 2, 1, 3)
          .reshape(_NW, nb, _CHUNK))

    mesh = plsc.VectorSubcoreMesh(core_axis_name="c", subcore_axis_name="s")
    body = functools.partial(_emb_body, nb, s, embed)
    out = pl.kernel(
        body,
        mesh=mesh,
        out_type=jax.ShapeDtypeStruct((n, embed), jnp.float32),
        scratch_types=[
            pltpu.VMEM((nb, _CHUNK), jnp.int32),
            pltpu.VMEM((_CHUNK, embed), jnp.float32),
            pltpu.VMEM((_CHUNK, embed), jnp.float32),
            pltpu.VMEM((_CHUNK, embed), jnp.float32),
            pltpu.VMEM((_CHUNK, embed), jnp.float32),
            pltpu.SemaphoreType.DMA,
            pltpu.SemaphoreType.DMA,
            pltpu.SemaphoreType.DMA,
            pltpu.SemaphoreType.DMA,
            pltpu.SemaphoreType.DMA,
            pltpu.SemaphoreType.DMA,
        ],
    )(xw, token_table, position_table)
    return out.reshape(b, s, embed)
